# baseline (speedup 1.0000x reference)
import jax
import jax.numpy as jnp
from jax import lax
from jax.experimental import pallas as pl
from jax.experimental.pallas import tpu as pltpu
from jax.experimental.pallas import tpu_sc as plsc

_N = 10000
_E = 320000
_NP = 10240
_EP = 327680
_NB1 = 160
_NB2 = 80
_RPT = _NP // 16
_RB = 256
_NRB = _NP // _RB

_mesh = plsc.VectorSubcoreMesh(core_axis_name="c", subcore_axis_name="s")


def _deg_body(dst32, ones_hbm, z128, degp, acc, onesv, dv, ssd):
    cid = lax.axis_index("c")
    sid = lax.axis_index("s")
    r0 = sid * _RPT
    pltpu.sync_copy(z128.at[pl.ds(r0, _RPT)], acc.at[pl.ds(r0, _RPT)])
    pltpu.sync_copy(ones_hbm, onesv)
    plsc.subcore_barrier()

    def body(i, carry):
        p = lax.rem(i, 2)
        pltpu.sync_copy(dst32.at[cid, sid, pl.ds(2 * i, 2)], dv.at[p])

        @pl.when(i > 0)
        def _():
            pltpu.make_async_copy(onesv, acc.at[dv.at[p, 0]], ssd).wait()
            pltpu.make_async_copy(onesv, acc.at[dv.at[p, 1]], ssd).wait()

        pltpu.async_copy(onesv, acc.at[dv.at[p, 0]], ssd, add=True)
        pltpu.async_copy(onesv, acc.at[dv.at[p, 1]], ssd, add=True)
        return carry

    lax.fori_loop(0, _NB2 // 2, body, 0)
    pltpu.make_async_copy(onesv, acc.at[dv.at[0, 0]], ssd).wait()
    pltpu.make_async_copy(onesv, acc.at[dv.at[0, 1]], ssd).wait()
    plsc.subcore_barrier()
    pltpu.sync_copy(acc.at[pl.ds(r0, _RPT)],
                    degp.at[pl.ds(cid * _NP + r0, _RPT)])


_deg_call = pl.kernel(
    _deg_body,
    out_type=jax.ShapeDtypeStruct((2 * _NP, 128), jnp.float32),
    mesh=_mesh,
    scratch_types=[
        pltpu.VMEM_SHARED((_NP, 128), jnp.float32),
        pltpu.VMEM((128, 128), jnp.float32),
        pltpu.VMEM((2, 2, 128), jnp.int32),
        pltpu.SemaphoreType.DMA,
    ],
)


def _prop_wide_body(a2flat, srcb, dst16, z128, uflat, acc,
                    sv, dv, rows0, rows1, sg0, sg1, ss0, ss1):
    cid = lax.axis_index("c")
    sid = lax.axis_index("s")
    r0 = sid * _RPT
    pltpu.sync_copy(z128.at[pl.ds(r0, _RPT)], acc.at[pl.ds(r0, _RPT)])
    plsc.subcore_barrier()

    def body(i, carry):
        p = lax.rem(i, 2)
        pltpu.sync_copy(srcb.at[cid, sid, pl.ds(2 * i, 2)], sv.at[p])
        pltpu.sync_copy(dst16.at[sid, pl.ds(2 * i, 2)], dv.at[p])

        @pl.when(i > 0)
        def _():
            pltpu.make_async_copy(rows0, acc.at[dv.at[p, 0]], ss0).wait()

        pltpu.async_copy(a2flat.at[sv.at[p, 0]], rows0, sg0)

        @pl.when(i > 0)
        def _():
            pltpu.make_async_copy(rows1, acc.at[dv.at[p, 1]], ss1).wait()

        pltpu.async_copy(a2flat.at[sv.at[p, 1]], rows1, sg1)
        pltpu.make_async_copy(a2flat.at[sv.at[p, 0]], rows0, sg0).wait()
        pltpu.async_copy(rows0, acc.at[dv.at[p, 0]], ss0, add=True)
        pltpu.make_async_copy(a2flat.at[sv.at[p, 1]], rows1, sg1).wait()
        pltpu.async_copy(rows1, acc.at[dv.at[p, 1]], ss1, add=True)
        return carry

    lax.fori_loop(0, _NB1 // 2, body, 0)
    pltpu.make_async_copy(rows0, acc.at[dv.at[0, 0]], ss0).wait()
    pltpu.make_async_copy(rows1, acc.at[dv.at[0, 1]], ss1).wait()
    plsc.subcore_barrier()
    pltpu.sync_copy(acc.at[pl.ds(r0, _RPT)],
                    uflat.at[pl.ds(cid * _NP + r0, _RPT)])


_prop_wide_call = pl.kernel(
    _prop_wide_body,
    out_type=jax.ShapeDtypeStruct((2 * _NP, 128), jnp.float32),
    mesh=_mesh,
    scratch_types=[
        pltpu.VMEM_SHARED((_NP, 128), jnp.float32),
        pltpu.VMEM((2, 2, 128), jnp.int32),
        pltpu.VMEM((2, 2, 128), jnp.int32),
        pltpu.VMEM((128, 128), jnp.float32),
        pltpu.VMEM((128, 128), jnp.float32),
        pltpu.SemaphoreType.DMA,
        pltpu.SemaphoreType.DMA,
        pltpu.SemaphoreType.DMA,
        pltpu.SemaphoreType.DMA,
    ],
)


def _prop_narrow_body(tbl, src32, dst32, z128, uflat, acc,
                      sv, dv, rows0, rows1, sg0, sg1, ss0, ss1):
    cid = lax.axis_index("c")
    sid = lax.axis_index("s")
    r0 = sid * _RPT
    pltpu.sync_copy(z128.at[pl.ds(r0, _RPT)], acc.at[pl.ds(r0, _RPT)])
    plsc.subcore_barrier()

    def body(i, carry):
        p = lax.rem(i, 2)
        pltpu.sync_copy(src32.at[cid, sid, pl.ds(2 * i, 2)], sv.at[p])
        pltpu.sync_copy(dst32.at[cid, sid, pl.ds(2 * i, 2)], dv.at[p])

        @pl.when(i > 0)
        def _():
            pltpu.make_async_copy(rows0, acc.at[dv.at[p, 0]], ss0).wait()

        pltpu.async_copy(tbl.at[sv.at[p, 0]], rows0, sg0)

        @pl.when(i > 0)
        def _():
            pltpu.make_async_copy(rows1, acc.at[dv.at[p, 1]], ss1).wait()

        pltpu.async_copy(tbl.at[sv.at[p, 1]], rows1, sg1)
        pltpu.make_async_copy(tbl.at[sv.at[p, 0]], rows0, sg0).wait()
        pltpu.async_copy(rows0, acc.at[dv.at[p, 0]], ss0, add=True)
        pltpu.make_async_copy(tbl.at[sv.at[p, 1]], rows1, sg1).wait()
        pltpu.async_copy(rows1, acc.at[dv.at[p, 1]], ss1, add=True)
        return carry

    lax.fori_loop(0, _NB2 // 2, body, 0)
    pltpu.make_async_copy(rows0, acc.at[dv.at[0, 0]], ss0).wait()
    pltpu.make_async_copy(rows1, acc.at[dv.at[0, 1]], ss1).wait()
    plsc.subcore_barrier()
    pltpu.sync_copy(acc.at[pl.ds(r0, _RPT)],
                    uflat.at[pl.ds(cid * _NP + r0, _RPT)])


_prop_narrow_call = pl.kernel(
    _prop_narrow_body,
    out_type=jax.ShapeDtypeStruct((2 * _NP, 128), jnp.float32),
    mesh=_mesh,
    scratch_types=[
        pltpu.VMEM_SHARED((_NP, 128), jnp.float32),
        pltpu.VMEM((2, 2, 128), jnp.int32),
        pltpu.VMEM((2, 2, 128), jnp.int32),
        pltpu.VMEM((128, 128), jnp.float32),
        pltpu.VMEM((128, 128), jnp.float32),
        pltpu.SemaphoreType.DMA,
        pltpu.SemaphoreType.DMA,
        pltpu.SemaphoreType.DMA,
        pltpu.SemaphoreType.DMA,
    ],
)


def _embed_body(x_ref, w_ref, b_ref, degp_ref, a_ref, dinv_ref):
    k = pl.program_id(0)
    i = pl.program_id(1)
    m = jnp.dot(x_ref[...], w_ref[...], preferred_element_type=jnp.float32)
    bb = jnp.where(k == 0, b_ref[0:1, :], b_ref[1:2, :])
    m = jnp.maximum(m + bb, 0.0)
    deg = degp_ref[0, :, 0] + degp_ref[1, :, 0]
    dinv = lax.rsqrt(jnp.maximum(deg, 1.0))
    rows = i * _RB + lax.broadcasted_iota(jnp.int32, (_RB,), 0)
    dinv = jnp.where(rows < _N, dinv, 0.0)
    a_ref[...] = (m * dinv[:, None])[None]
    dinv_ref[...] = jnp.broadcast_to(dinv[:, None], (_RB, 16))


def _embed(xpad, w, b2, degp):
    return pl.pallas_call(
        _embed_body,
        grid=(2, _NRB),
        in_specs=[
            pl.BlockSpec((_RB, 128), lambda k, i: (i, 0)),
            pl.BlockSpec((128, 128), lambda k, i: (0, k)),
            pl.BlockSpec((2, 128), lambda k, i: (0, 0)),
            pl.BlockSpec((2, _RB, 128), lambda k, i: (0, i, 0)),
        ],
        out_specs=[
            pl.BlockSpec((1, _RB, 128), lambda k, i: (k, i, 0)),
            pl.BlockSpec((_RB, 16), lambda k, i: (i, 0)),
        ],
        out_shape=[
            jax.ShapeDtypeStruct((2, _NP, 128), jnp.float32),
            jax.ShapeDtypeStruct((_NP, 16), jnp.float32),
        ],
    )(xpad, w, b2, degp)


def _stats_body(u_ref, dinv_ref, b_ref, s_ref, q_ref):
    i = pl.program_id(1)
    dinv = dinv_ref[:, 0]
    h = u_ref[0] * dinv[:, None]
    b_ref[...] = (h * dinv[:, None])[None]
    ps = jnp.broadcast_to(jnp.sum(h, axis=0)[None, None], (1, 8, 128))
    pq = jnp.broadcast_to(jnp.sum(h * h, axis=0)[None, None], (1, 8, 128))

    @pl.when(i == 0)
    def _():
        s_ref[...] = ps
        q_ref[...] = pq

    @pl.when(i > 0)
    def _():
        s_ref[...] += ps
        q_ref[...] += pq


def _stats_scale(u1, dinvp):
    return pl.pallas_call(
        _stats_body,
        grid=(2, _NRB),
        in_specs=[
            pl.BlockSpec((1, _RB, 128), lambda k, i: (k, i, 0)),
            pl.BlockSpec((_RB, 16), lambda k, i: (i, 0)),
        ],
        out_specs=[
            pl.BlockSpec((1, _RB, 128), lambda k, i: (k, i, 0)),
            pl.BlockSpec((1, 8, 128), lambda k, i: (k, 0, 0)),
            pl.BlockSpec((1, 8, 128), lambda k, i: (k, 0, 0)),
        ],
        out_shape=[
            jax.ShapeDtypeStruct((2, _NP, 128), jnp.float32),
            jax.ShapeDtypeStruct((2, 8, 128), jnp.float32),
            jax.ShapeDtypeStruct((2, 8, 128), jnp.float32),
        ],
    )(u1, dinvp)


def _stats2_body(u_ref, dinv_ref, s_ref, q_ref):
    i = pl.program_id(1)
    dinv = dinv_ref[:, 0]
    h = u_ref[0] * dinv[:, None]
    ps = jnp.broadcast_to(jnp.sum(h, axis=0)[None, None], (1, 8, 128))
    pq = jnp.broadcast_to(jnp.sum(h * h, axis=0)[None, None], (1, 8, 128))

    @pl.when(i == 0)
    def _():
        s_ref[...] = ps
        q_ref[...] = pq

    @pl.when(i > 0)
    def _():
        s_ref[...] += ps
        q_ref[...] += pq


def _stats_only(u2, dinvp):
    return pl.pallas_call(
        _stats2_body,
        grid=(2, _NRB),
        in_specs=[
            pl.BlockSpec((1, _RB, 128), lambda k, i: (k, i, 0)),
            pl.BlockSpec((_RB, 16), lambda k, i: (i, 0)),
        ],
        out_specs=[
            pl.BlockSpec((1, 8, 128), lambda k, i: (k, 0, 0)),
            pl.BlockSpec((1, 8, 128), lambda k, i: (k, 0, 0)),
        ],
        out_shape=[
            jax.ShapeDtypeStruct((2, 8, 128), jnp.float32),
            jax.ShapeDtypeStruct((2, 8, 128), jnp.float32),
        ],
    )(u2, dinvp)


def _bn_body(u1_ref, u2_ref, dinv_ref, s1_ref, q1_ref, s2_ref, q2_ref,
             gam_ref, bet_ref, wo_ref, g1s_ref, p2_ref):
    dinv = dinv_ref[:, 0]
    acc1 = jnp.zeros((_RB, 16), jnp.float32)
    acc2 = jnp.zeros((_RB, 16), jnp.float32)
    for part, (u_ref, s_ref, q_ref) in enumerate(
            [(u1_ref, s1_ref, q1_ref), (u2_ref, s2_ref, q2_ref)]):
        for c in range(2):
            h = u_ref[c] * dinv[:, None]
            mu = s_ref[c, 0] * (1.0 / _N)
            var = q_ref[c, 0] * (1.0 / _N) - mu * mu
            rstd = lax.rsqrt(var + 1e-5)
            g = gam_ref[part * 2 + c]
            bb = bet_ref[part * 2 + c]
            hb = jnp.maximum((h - mu[None]) * rstd[None] * g[None] + bb[None],
                             0.0)
            r0w = part * 256 + c * 128
            acc1 += jnp.dot(hb, wo_ref[r0w:r0w + 128, :],
                            preferred_element_type=jnp.float32)
            acc2 += jnp.dot(hb, wo_ref[512 + r0w:512 + r0w + 128, :],
                            preferred_element_type=jnp.float32)
    zpad = jnp.zeros((_RB, 112), jnp.float32)
    g1s_ref[...] = jnp.concatenate([acc1 * dinv[:, None], zpad], axis=1)
    p2_ref[...] = jnp.concatenate([acc2 * dinv[:, None], zpad], axis=1)


def _bn_matmul(u1, u2, dinvp, s1, q1, s2, q2, gam2, bet2, wo):
    return pl.pallas_call(
        _bn_body,
        grid=(_NRB,),
        in_specs=[
            pl.BlockSpec((2, _RB, 128), lambda i: (0, i, 0)),
            pl.BlockSpec((2, _RB, 128), lambda i: (0, i, 0)),
            pl.BlockSpec((_RB, 16), lambda i: (i, 0)),
            pl.BlockSpec((2, 8, 128), lambda i: (0, 0, 0)),
            pl.BlockSpec((2, 8, 128), lambda i: (0, 0, 0)),
            pl.BlockSpec((2, 8, 128), lambda i: (0, 0, 0)),
            pl.BlockSpec((2, 8, 128), lambda i: (0, 0, 0)),
            pl.BlockSpec((4, 128), lambda i: (0, 0)),
            pl.BlockSpec((4, 128), lambda i: (0, 0)),
            pl.BlockSpec((1024, 16), lambda i: (0, 0)),
        ],
        out_specs=[
            pl.BlockSpec((_RB, 128), lambda i: (i, 0)),
            pl.BlockSpec((_RB, 128), lambda i: (i, 0)),
        ],
        out_shape=[
            jax.ShapeDtypeStruct((_NP, 128), jnp.float32),
            jax.ShapeDtypeStruct((_NP, 128), jnp.float32),
        ],
    )(u1, u2, dinvp, s1, q1, s2, q2, gam2, bet2, wo)


def _mid_body(g1s_ref, u3_ref, dinv_ref, q_ref):
    d = dinv_ref[:, 0:1]
    q_ref[...] = g1s_ref[...] + d * d * (u3_ref[0] + u3_ref[1])


def _mid(g1s, u3, dinvp):
    return pl.pallas_call(
        _mid_body,
        grid=(_NRB,),
        in_specs=[
            pl.BlockSpec((_RB, 128), lambda i: (i, 0)),
            pl.BlockSpec((2, _RB, 128), lambda i: (0, i, 0)),
            pl.BlockSpec((_RB, 16), lambda i: (i, 0)),
        ],
        out_specs=pl.BlockSpec((_RB, 128), lambda i: (i, 0)),
        out_shape=jax.ShapeDtypeStruct((_NP, 128), jnp.float32),
    )(g1s, u3, dinvp)


def _final_body(u4_ref, dinv_ref, bout_ref, out_ref):
    d = dinv_ref[:, 0:1]
    s = u4_ref[0, :, :16] + u4_ref[1, :, :16]
    out_ref[...] = d * s + bout_ref[...]


def _final(u4, dinvp, bout):
    return pl.pallas_call(
        _final_body,
        grid=(_NRB,),
        in_specs=[
            pl.BlockSpec((2, _RB, 128), lambda i: (0, i, 0)),
            pl.BlockSpec((_RB, 16), lambda i: (i, 0)),
            pl.BlockSpec((1, 16), lambda i: (0, 0)),
        ],
        out_specs=pl.BlockSpec((_RB, 16), lambda i: (i, 0)),
        out_shape=jax.ShapeDtypeStruct((_NP, 16), jnp.float32),
    )(u4, dinvp, bout)


def kernel(x, edge_index, W_embed, b_embed, bn_gamma, bn_beta, W_out, b_out):
    src = edge_index[0].astype(jnp.int32)
    dst = edge_index[1].astype(jnp.int32)
    pad = jnp.full((_EP - _E,), _N, jnp.int32)
    srcp = jnp.concatenate([src, pad])
    dstp = jnp.concatenate([dst, pad])
    src16b = (srcp[None, :]
              + (jnp.arange(2, dtype=jnp.int32) * _NP)[:, None]
              ).reshape(2, 16, _NB1, 128)
    dst16 = dstp.reshape(16, _NB1, 128)
    src32 = srcp.reshape(2, 16, _NB2, 128)
    dst32 = dstp.reshape(2, 16, _NB2, 128)

    xpad = jnp.pad(x, ((0, _NP - _N), (0, 0)))
    z128 = jnp.zeros((_NP, 128), jnp.float32)
    ones128 = jnp.zeros((128, 128), jnp.float32).at[:, 0].set(1.0)
    b2 = b_embed.reshape(2, 128)
    gam2 = bn_gamma.reshape(4, 128)
    bet2 = bn_beta.reshape(4, 128)
    bout = b_out.reshape(1, 16)

    degp = _deg_call(dst32, ones128, z128).reshape(2, _NP, 128)
    a2, dinvp = _embed(xpad, W_embed, b2, degp)

    u1 = _prop_wide_call(a2.reshape(2 * _NP, 128), src16b, dst16, z128)
    u1 = u1.reshape(2, _NP, 128)
    b_in, s1, q1 = _stats_scale(u1, dinvp)
    u2 = _prop_wide_call(b_in.reshape(2 * _NP, 128), src16b, dst16, z128)
    u2 = u2.reshape(2, _NP, 128)
    s2, q2 = _stats_only(u2, dinvp)

    g1s, p2 = _bn_matmul(u1, u2, dinvp, s1, q1, s2, q2, gam2, bet2, W_out)

    u3 = _prop_narrow_call(p2, src32, dst32, z128).reshape(2, _NP, 128)
    qp = _mid(g1s, u3, dinvp)
    u4 = _prop_narrow_call(qp, src32, dst32, z128).reshape(2, _NP, 128)
    logits = _final(u4, dinvp, bout)
    return logits[:_N]


# speedup vs baseline: 2.1976x; 2.1976x over previous
import jax
import jax.numpy as jnp
from jax import lax
from jax.experimental import pallas as pl
from jax.experimental.pallas import tpu as pltpu
from jax.experimental.pallas import tpu_sc as plsc

_N = 10000
_E = 320000
_NP = 10240
_EP = 327680
_NB1 = 160
_NB2 = 80
_RPT = _NP // 16
_RB = 256
_NRB = _NP // _RB

_mesh = plsc.VectorSubcoreMesh(core_axis_name="c", subcore_axis_name="s")


def _deg_body(dst32, ones_hbm, z128, degp, acc, onesv, dv, ssd):
    cid = lax.axis_index("c")
    sid = lax.axis_index("s")
    r0 = sid * _RPT
    pltpu.sync_copy(z128.at[pl.ds(r0, _RPT)], acc.at[pl.ds(r0, _RPT)])
    pltpu.sync_copy(ones_hbm, onesv)
    plsc.subcore_barrier()

    def body(i, carry):
        p = lax.rem(i, 2)
        pltpu.sync_copy(dst32.at[cid, sid, pl.ds(2 * i, 2)], dv.at[p])

        @pl.when(i > 0)
        def _():
            pltpu.make_async_copy(onesv, acc.at[dv.at[p, 0]], ssd).wait()
            pltpu.make_async_copy(onesv, acc.at[dv.at[p, 1]], ssd).wait()

        pltpu.async_copy(onesv, acc.at[dv.at[p, 0]], ssd, add=True)
        pltpu.async_copy(onesv, acc.at[dv.at[p, 1]], ssd, add=True)
        return carry

    lax.fori_loop(0, _NB2 // 2, body, 0)
    pltpu.make_async_copy(onesv, acc.at[dv.at[0, 0]], ssd).wait()
    pltpu.make_async_copy(onesv, acc.at[dv.at[0, 1]], ssd).wait()
    plsc.subcore_barrier()
    pltpu.sync_copy(acc.at[pl.ds(r0, _RPT)],
                    degp.at[pl.ds(cid * _NP + r0, _RPT)])


_deg_call = pl.kernel(
    _deg_body,
    out_type=jax.ShapeDtypeStruct((2 * _NP, 128), jnp.float32),
    mesh=_mesh,
    scratch_types=[
        pltpu.VMEM_SHARED((_NP, 128), jnp.float32),
        pltpu.VMEM((128, 128), jnp.float32),
        pltpu.VMEM((2, 2, 128), jnp.int32),
        pltpu.SemaphoreType.DMA,
    ],
)


def _prop_wide_body(a2flat, srcb, dst16, z128, uflat, acc,
                    sv, dv, rows0, rows1, sg0, sg1, ss0, ss1):
    cid = lax.axis_index("c")
    sid = lax.axis_index("s")
    r0 = sid * _RPT
    pltpu.sync_copy(z128.at[pl.ds(r0, _RPT)], acc.at[pl.ds(r0, _RPT)])
    plsc.subcore_barrier()

    def body(i, carry):
        p = lax.rem(i, 2)
        pltpu.sync_copy(srcb.at[cid, sid, pl.ds(2 * i, 2)], sv.at[p])
        pltpu.sync_copy(dst16.at[sid, pl.ds(2 * i, 2)], dv.at[p])

        @pl.when(i > 0)
        def _():
            pltpu.make_async_copy(rows0, acc.at[dv.at[p, 0]], ss0).wait()

        pltpu.async_copy(a2flat.at[sv.at[p, 0]], rows0, sg0)

        @pl.when(i > 0)
        def _():
            pltpu.make_async_copy(rows1, acc.at[dv.at[p, 1]], ss1).wait()

        pltpu.async_copy(a2flat.at[sv.at[p, 1]], rows1, sg1)
        pltpu.make_async_copy(a2flat.at[sv.at[p, 0]], rows0, sg0).wait()
        pltpu.async_copy(rows0, acc.at[dv.at[p, 0]], ss0, add=True)
        pltpu.make_async_copy(a2flat.at[sv.at[p, 1]], rows1, sg1).wait()
        pltpu.async_copy(rows1, acc.at[dv.at[p, 1]], ss1, add=True)
        return carry

    lax.fori_loop(0, _NB1 // 2, body, 0)
    pltpu.make_async_copy(rows0, acc.at[dv.at[0, 0]], ss0).wait()
    pltpu.make_async_copy(rows1, acc.at[dv.at[0, 1]], ss1).wait()
    plsc.subcore_barrier()
    pltpu.sync_copy(acc.at[pl.ds(r0, _RPT)],
                    uflat.at[pl.ds(cid * _NP + r0, _RPT)])


_prop_wide_call = pl.kernel(
    _prop_wide_body,
    out_type=jax.ShapeDtypeStruct((2 * _NP, 128), jnp.float32),
    mesh=_mesh,
    scratch_types=[
        pltpu.VMEM_SHARED((_NP, 128), jnp.float32),
        pltpu.VMEM((2, 2, 128), jnp.int32),
        pltpu.VMEM((2, 2, 128), jnp.int32),
        pltpu.VMEM((128, 128), jnp.float32),
        pltpu.VMEM((128, 128), jnp.float32),
        pltpu.SemaphoreType.DMA,
        pltpu.SemaphoreType.DMA,
        pltpu.SemaphoreType.DMA,
        pltpu.SemaphoreType.DMA,
    ],
)


def _prop_narrow_body(tbl, src32, dst32, z128, uflat, acc,
                      sv, dv, rows0, rows1, sg0, sg1, ss0, ss1):
    cid = lax.axis_index("c")
    sid = lax.axis_index("s")
    r0 = sid * _RPT
    pltpu.sync_copy(z128.at[pl.ds(r0, _RPT)], acc.at[pl.ds(r0, _RPT)])
    plsc.subcore_barrier()

    def body(i, carry):
        p = lax.rem(i, 2)
        pltpu.sync_copy(src32.at[cid, sid, pl.ds(2 * i, 2)], sv.at[p])
        pltpu.sync_copy(dst32.at[cid, sid, pl.ds(2 * i, 2)], dv.at[p])

        @pl.when(i > 0)
        def _():
            pltpu.make_async_copy(rows0, acc.at[dv.at[p, 0]], ss0).wait()

        pltpu.async_copy(tbl.at[sv.at[p, 0]], rows0, sg0)

        @pl.when(i > 0)
        def _():
            pltpu.make_async_copy(rows1, acc.at[dv.at[p, 1]], ss1).wait()

        pltpu.async_copy(tbl.at[sv.at[p, 1]], rows1, sg1)
        pltpu.make_async_copy(tbl.at[sv.at[p, 0]], rows0, sg0).wait()
        pltpu.async_copy(rows0, acc.at[dv.at[p, 0]], ss0, add=True)
        pltpu.make_async_copy(tbl.at[sv.at[p, 1]], rows1, sg1).wait()
        pltpu.async_copy(rows1, acc.at[dv.at[p, 1]], ss1, add=True)
        return carry

    lax.fori_loop(0, _NB2 // 2, body, 0)
    pltpu.make_async_copy(rows0, acc.at[dv.at[0, 0]], ss0).wait()
    pltpu.make_async_copy(rows1, acc.at[dv.at[0, 1]], ss1).wait()
    plsc.subcore_barrier()
    pltpu.sync_copy(acc.at[pl.ds(r0, _RPT)],
                    uflat.at[pl.ds(cid * _NP + r0, _RPT)])


_prop_narrow_call = pl.kernel(
    _prop_narrow_body,
    out_type=jax.ShapeDtypeStruct((2 * _NP, 128), jnp.float32),
    mesh=_mesh,
    scratch_types=[
        pltpu.VMEM_SHARED((_NP, 128), jnp.float32),
        pltpu.VMEM((2, 2, 128), jnp.int32),
        pltpu.VMEM((2, 2, 128), jnp.int32),
        pltpu.VMEM((128, 128), jnp.float32),
        pltpu.VMEM((128, 128), jnp.float32),
        pltpu.SemaphoreType.DMA,
        pltpu.SemaphoreType.DMA,
        pltpu.SemaphoreType.DMA,
        pltpu.SemaphoreType.DMA,
    ],
)


def _embed_body(x_ref, w_ref, b_ref, degp_ref, a_ref, dinv_ref):
    k = pl.program_id(0)
    i = pl.program_id(1)
    m = jnp.dot(x_ref[...], w_ref[...], preferred_element_type=jnp.float32)
    bb = jnp.where(k == 0, b_ref[0:1, :], b_ref[1:2, :])
    m = jnp.maximum(m + bb, 0.0)
    deg = degp_ref[0, :, 0] + degp_ref[1, :, 0]
    dinv = lax.rsqrt(jnp.maximum(deg, 1.0))
    rows = i * _RB + lax.broadcasted_iota(jnp.int32, (_RB,), 0)
    dinv = jnp.where(rows < _N, dinv, 0.0)
    a_ref[...] = (m * dinv[:, None])[None]
    dinv_ref[...] = jnp.broadcast_to(dinv[:, None], (_RB, 16))


def _embed(xpad, w, b2, degp):
    return pl.pallas_call(
        _embed_body,
        grid=(2, _NRB),
        in_specs=[
            pl.BlockSpec((_RB, 128), lambda k, i: (i, 0)),
            pl.BlockSpec((128, 128), lambda k, i: (0, k)),
            pl.BlockSpec((2, 128), lambda k, i: (0, 0)),
            pl.BlockSpec((2, _RB, 128), lambda k, i: (0, i, 0)),
        ],
        out_specs=[
            pl.BlockSpec((1, _RB, 128), lambda k, i: (k, i, 0)),
            pl.BlockSpec((_RB, 16), lambda k, i: (i, 0)),
        ],
        out_shape=[
            jax.ShapeDtypeStruct((2, _NP, 128), jnp.float32),
            jax.ShapeDtypeStruct((_NP, 16), jnp.float32),
        ],
    )(xpad, w, b2, degp)


def _stats_body(u_ref, dinv_ref, b_ref, s_ref, q_ref):
    i = pl.program_id(1)
    dinv = dinv_ref[:, 0]
    h = u_ref[0] * dinv[:, None]
    b_ref[...] = (h * dinv[:, None])[None]
    ps = jnp.broadcast_to(jnp.sum(h, axis=0)[None, None], (1, 8, 128))
    pq = jnp.broadcast_to(jnp.sum(h * h, axis=0)[None, None], (1, 8, 128))

    @pl.when(i == 0)
    def _():
        s_ref[...] = ps
        q_ref[...] = pq

    @pl.when(i > 0)
    def _():
        s_ref[...] += ps
        q_ref[...] += pq


def _stats_scale(u1, dinvp):
    return pl.pallas_call(
        _stats_body,
        grid=(2, _NRB),
        in_specs=[
            pl.BlockSpec((1, _RB, 128), lambda k, i: (k, i, 0)),
            pl.BlockSpec((_RB, 16), lambda k, i: (i, 0)),
        ],
        out_specs=[
            pl.BlockSpec((1, _RB, 128), lambda k, i: (k, i, 0)),
            pl.BlockSpec((1, 8, 128), lambda k, i: (k, 0, 0)),
            pl.BlockSpec((1, 8, 128), lambda k, i: (k, 0, 0)),
        ],
        out_shape=[
            jax.ShapeDtypeStruct((2, _NP, 128), jnp.float32),
            jax.ShapeDtypeStruct((2, 8, 128), jnp.float32),
            jax.ShapeDtypeStruct((2, 8, 128), jnp.float32),
        ],
    )(u1, dinvp)


def _stats2_body(u_ref, dinv_ref, s_ref, q_ref):
    i = pl.program_id(1)
    dinv = dinv_ref[:, 0]
    h = u_ref[0] * dinv[:, None]
    ps = jnp.broadcast_to(jnp.sum(h, axis=0)[None, None], (1, 8, 128))
    pq = jnp.broadcast_to(jnp.sum(h * h, axis=0)[None, None], (1, 8, 128))

    @pl.when(i == 0)
    def _():
        s_ref[...] = ps
        q_ref[...] = pq

    @pl.when(i > 0)
    def _():
        s_ref[...] += ps
        q_ref[...] += pq


def _stats_only(u2, dinvp):
    return pl.pallas_call(
        _stats2_body,
        grid=(2, _NRB),
        in_specs=[
            pl.BlockSpec((1, _RB, 128), lambda k, i: (k, i, 0)),
            pl.BlockSpec((_RB, 16), lambda k, i: (i, 0)),
        ],
        out_specs=[
            pl.BlockSpec((1, 8, 128), lambda k, i: (k, 0, 0)),
            pl.BlockSpec((1, 8, 128), lambda k, i: (k, 0, 0)),
        ],
        out_shape=[
            jax.ShapeDtypeStruct((2, 8, 128), jnp.float32),
            jax.ShapeDtypeStruct((2, 8, 128), jnp.float32),
        ],
    )(u2, dinvp)


def _bn_body(u1_ref, u2_ref, dinv_ref, s1_ref, q1_ref, s2_ref, q2_ref,
             gam_ref, bet_ref, wo_ref, g1s_ref, p2_ref):
    dinv = dinv_ref[:, 0]
    acc1 = jnp.zeros((_RB, 16), jnp.float32)
    acc2 = jnp.zeros((_RB, 16), jnp.float32)
    for part, (u_ref, s_ref, q_ref) in enumerate(
            [(u1_ref, s1_ref, q1_ref), (u2_ref, s2_ref, q2_ref)]):
        for c in range(2):
            h = u_ref[c] * dinv[:, None]
            mu = s_ref[c, 0] * (1.0 / _N)
            var = q_ref[c, 0] * (1.0 / _N) - mu * mu
            rstd = lax.rsqrt(var + 1e-5)
            g = gam_ref[part * 2 + c]
            bb = bet_ref[part * 2 + c]
            hb = jnp.maximum((h - mu[None]) * rstd[None] * g[None] + bb[None],
                             0.0)
            r0w = part * 256 + c * 128
            acc1 += jnp.dot(hb, wo_ref[r0w:r0w + 128, :],
                            preferred_element_type=jnp.float32)
            acc2 += jnp.dot(hb, wo_ref[512 + r0w:512 + r0w + 128, :],
                            preferred_element_type=jnp.float32)
    zpad = jnp.zeros((_RB, 112), jnp.float32)
    g1s_ref[...] = jnp.concatenate([acc1 * dinv[:, None], zpad], axis=1)
    p2_ref[...] = jnp.concatenate([acc2 * dinv[:, None], zpad], axis=1)


def _bn_matmul(u1, u2, dinvp, s1, q1, s2, q2, gam2, bet2, wo):
    return pl.pallas_call(
        _bn_body,
        grid=(_NRB,),
        in_specs=[
            pl.BlockSpec((2, _RB, 128), lambda i: (0, i, 0)),
            pl.BlockSpec((2, _RB, 128), lambda i: (0, i, 0)),
            pl.BlockSpec((_RB, 16), lambda i: (i, 0)),
            pl.BlockSpec((2, 8, 128), lambda i: (0, 0, 0)),
            pl.BlockSpec((2, 8, 128), lambda i: (0, 0, 0)),
            pl.BlockSpec((2, 8, 128), lambda i: (0, 0, 0)),
            pl.BlockSpec((2, 8, 128), lambda i: (0, 0, 0)),
            pl.BlockSpec((4, 128), lambda i: (0, 0)),
            pl.BlockSpec((4, 128), lambda i: (0, 0)),
            pl.BlockSpec((1024, 16), lambda i: (0, 0)),
        ],
        out_specs=[
            pl.BlockSpec((_RB, 128), lambda i: (i, 0)),
            pl.BlockSpec((_RB, 128), lambda i: (i, 0)),
        ],
        out_shape=[
            jax.ShapeDtypeStruct((_NP, 128), jnp.float32),
            jax.ShapeDtypeStruct((_NP, 128), jnp.float32),
        ],
    )(u1, u2, dinvp, s1, q1, s2, q2, gam2, bet2, wo)


def _mid_body(g1s_ref, u3_ref, dinv_ref, q_ref):
    d = dinv_ref[:, 0:1]
    q_ref[...] = g1s_ref[...] + d * d * (u3_ref[0] + u3_ref[1])


def _mid(g1s, u3, dinvp):
    return pl.pallas_call(
        _mid_body,
        grid=(_NRB,),
        in_specs=[
            pl.BlockSpec((_RB, 128), lambda i: (i, 0)),
            pl.BlockSpec((2, _RB, 128), lambda i: (0, i, 0)),
            pl.BlockSpec((_RB, 16), lambda i: (i, 0)),
        ],
        out_specs=pl.BlockSpec((_RB, 128), lambda i: (i, 0)),
        out_shape=jax.ShapeDtypeStruct((_NP, 128), jnp.float32),
    )(g1s, u3, dinvp)


def _final_body(u4_ref, dinv_ref, bout_ref, out_ref):
    d = dinv_ref[:, 0:1]
    s = u4_ref[0, :, :16] + u4_ref[1, :, :16]
    out_ref[...] = d * s + bout_ref[...]


def _final(u4, dinvp, bout):
    return pl.pallas_call(
        _final_body,
        grid=(_NRB,),
        in_specs=[
            pl.BlockSpec((2, _RB, 128), lambda i: (0, i, 0)),
            pl.BlockSpec((_RB, 16), lambda i: (i, 0)),
            pl.BlockSpec((1, 16), lambda i: (0, 0)),
        ],
        out_specs=pl.BlockSpec((_RB, 16), lambda i: (i, 0)),
        out_shape=jax.ShapeDtypeStruct((_NP, 16), jnp.float32),
    )(u4, dinvp, bout)


def kernel(x, edge_index, W_embed, b_embed, bn_gamma, bn_beta, W_out, b_out):
    src = edge_index[0].astype(jnp.int32)
    dst = edge_index[1].astype(jnp.int32)
    pad = _N + (jnp.arange(_EP - _E, dtype=jnp.int32) % (_NP - _N))
    srcp = jnp.concatenate([src, pad])
    dstp = jnp.concatenate([dst, pad])
    src16b = (srcp[None, :]
              + (jnp.arange(2, dtype=jnp.int32) * _NP)[:, None]
              ).reshape(2, 16, _NB1, 128)
    dst16 = dstp.reshape(16, _NB1, 128)
    src32 = srcp.reshape(2, 16, _NB2, 128)
    dst32 = dstp.reshape(2, 16, _NB2, 128)

    xpad = jnp.pad(x, ((0, _NP - _N), (0, 0)))
    z128 = jnp.zeros((_NP, 128), jnp.float32)
    ones128 = jnp.zeros((128, 128), jnp.float32).at[:, 0].set(1.0)
    b2 = b_embed.reshape(2, 128)
    gam2 = bn_gamma.reshape(4, 128)
    bet2 = bn_beta.reshape(4, 128)
    bout = b_out.reshape(1, 16)

    degp = _deg_call(dst32, ones128, z128).reshape(2, _NP, 128)
    a2, dinvp = _embed(xpad, W_embed, b2, degp)

    u1 = _prop_wide_call(a2.reshape(2 * _NP, 128), src16b, dst16, z128)
    u1 = u1.reshape(2, _NP, 128)
    b_in, s1, q1 = _stats_scale(u1, dinvp)
    u2 = _prop_wide_call(b_in.reshape(2 * _NP, 128), src16b, dst16, z128)
    u2 = u2.reshape(2, _NP, 128)
    s2, q2 = _stats_only(u2, dinvp)

    g1s, p2 = _bn_matmul(u1, u2, dinvp, s1, q1, s2, q2, gam2, bet2, W_out)

    u3 = _prop_narrow_call(p2, src32, dst32, z128).reshape(2, _NP, 128)
    qp = _mid(g1s, u3, dinvp)
    u4 = _prop_narrow_call(qp, src32, dst32, z128).reshape(2, _NP, 128)
    logits = _final(u4, dinvp, bout)
    return logits[:_N]
